# Initial kernel scaffold; baseline (speedup 1.0000x reference)
import jax
import jax.numpy as jnp
from jax.experimental import pallas as pl


def kernel(preds, bound_args, direction, ps_sorted_objs, ps_offsets, po_sorted_subjs, po_offsets):
    raise NotImplementedError("write your pallas kernel here")


# speedup vs baseline: 423.7355x; 423.7355x over previous
import functools

import jax
import jax.numpy as jnp
from jax import lax
from jax.experimental import pallas as pl
from jax.experimental.pallas import tpu as pltpu
from jax.experimental.pallas import tpu_sc as plsc

P = 32
E = 50000
F = 1600000
M = 64
N = 65536
PE1 = P * E + 1
FP = F + M
FLAT_LEN = 2 * FP + M
G_ROWS = 50008
OFF_LEN = 2 * PE1
OFF_PAD = 3200008

NC = 2
NS = 16
NW = NC * NS
QPW = N // NW
CH = 64
NCHUNK = QPW // CH
LANES = 16
NG = CH // LANES


def _sc_enumerate(preds32, bounds32, dirs32, off_cat, gtab):
    mesh = plsc.VectorSubcoreMesh(core_axis_name="c", subcore_axis_name="s")

    @functools.partial(
        pl.kernel,
        out_type=[
            jax.ShapeDtypeStruct((N, M), jnp.int32),
            jax.ShapeDtypeStruct((N,), jnp.int32),
        ],
        mesh=mesh,
        compiler_params=pltpu.CompilerParams(needs_layout_passes=False),
        scratch_types=[
            pltpu.VMEM((QPW,), jnp.int32),
            pltpu.VMEM((QPW,), jnp.int32),
            pltpu.VMEM((QPW,), jnp.int32),
            pltpu.VMEM((2 * CH,), jnp.int32),
            pltpu.VMEM((2 * CH,), jnp.int32),
            pltpu.VMEM((CH,), jnp.int32),
            pltpu.VMEM((CH,), jnp.int32),
            pltpu.VMEM((CH,), jnp.int32),
            pltpu.VMEM((CH, 2 * M), jnp.int32),
            pltpu.VMEM((CH, M), jnp.int32),
            pltpu.SemaphoreType.DMA,
        ],
    )
    def body(p_hbm, b_hbm, d_hbm, off_hbm, g_hbm, cand_hbm, cnt_hbm,
             pv, bv, dv, idx_off, so_buf, rows_idx, counts_v, addr_v,
             buf, out_v, sem):
        wid = lax.axis_index("s") * NC + lax.axis_index("c")
        qw = wid * jnp.int32(QPW)
        pltpu.sync_copy(p_hbm.at[pl.ds(qw, QPW)], pv)
        pltpu.sync_copy(b_hbm.at[pl.ds(qw, QPW)], bv)
        pltpu.sync_copy(d_hbm.at[pl.ds(qw, QPW)], dv)
        iota = lax.iota(jnp.int32, LANES)

        def chunk(c, carry):
            cb = c * jnp.int32(CH)
            for g in range(NG):
                s = pl.ds(cb + g * LANES, LANES)
                k2 = dv[s] * jnp.int32(PE1) + pv[s] * jnp.int32(E) + bv[s]
                idx_off[pl.ds(g * LANES, LANES)] = k2
                idx_off[pl.ds(CH + g * LANES, LANES)] = k2 + jnp.int32(1)
            pltpu.async_copy(off_hbm.at[idx_off], so_buf, sem).wait()
            for g in range(NG):
                st = so_buf[pl.ds(g * LANES, LANES)]
                en = so_buf[pl.ds(CH + g * LANES, LANES)]
                cnt = jnp.minimum(jnp.maximum(en - st, jnp.int32(0)),
                                  jnp.int32(M))
                counts_v[pl.ds(g * LANES, LANES)] = cnt
                g0 = dv[pl.ds(cb + g * LANES, LANES)] * jnp.int32(FP) + st
                lq = iota + jnp.int32(g * LANES)
                rows_idx[pl.ds(g * LANES, LANES)] = (
                    lax.shift_right_logical(g0, jnp.int32(6)))
                addr_v[pl.ds(g * LANES, LANES)] = (
                    lq * jnp.int32(2 * M) + lax.bitwise_and(g0, jnp.int32(M - 1)))
            pltpu.async_copy(g_hbm.at[rows_idx], buf, sem).wait()
            for g in range(NG):
                lq = iota + jnp.int32(g * LANES)

                def jl(j, acc):
                    a, cj = acc
                    r = lax.shift_right_logical(a, jnp.int32(7))
                    col = lax.bitwise_and(a, jnp.int32(2 * M - 1))
                    vals = plsc.load_gather(buf, [r, col])
                    plsc.store_scatter(out_v, [lq, cj], vals)
                    return (a + jnp.int32(1), cj + jnp.int32(1))

                lax.fori_loop(0, M, jl,
                              (addr_v[pl.ds(g * LANES, LANES)],
                               jnp.zeros((LANES,), jnp.int32)),
                              unroll=8)
            pltpu.sync_copy(out_v, cand_hbm.at[pl.ds(qw + cb, CH)])
            pltpu.sync_copy(counts_v, cnt_hbm.at[pl.ds(qw + cb, CH)])
            return carry

        lax.fori_loop(jnp.int32(0), jnp.int32(NCHUNK), chunk, jnp.int32(0))

    return body(preds32, bounds32, dirs32, off_cat, gtab)


def kernel(preds, bound_args, direction, ps_sorted_objs, ps_offsets,
           po_sorted_subjs, po_offsets):
    p32 = preds.astype(jnp.int32)
    b32 = bound_args.astype(jnp.int32)
    d32 = direction.astype(jnp.int32)
    off_cat = jnp.concatenate([
        ps_offsets.astype(jnp.int32),
        po_offsets.astype(jnp.int32),
        jnp.zeros((OFF_PAD - OFF_LEN,), jnp.int32),
    ])
    ps32 = ps_sorted_objs.astype(jnp.int32)
    po32 = po_sorted_subjs.astype(jnp.int32)
    flat = jnp.concatenate([
        ps32, jnp.broadcast_to(ps32[-1], (M,)),
        po32, jnp.broadcast_to(po32[-1], (M,)),
        jnp.zeros((M,), jnp.int32),
    ])
    base = flat.reshape(-1, M)
    g2 = jnp.concatenate([base[:-1], base[1:]], axis=1)
    gtab = jnp.concatenate(
        [g2, jnp.zeros((G_ROWS - g2.shape[0], 2 * M), jnp.int32)])
    cand32, counts = _sc_enumerate(p32, b32, d32, off_cat, gtab)
    candidates = cand32.astype(jnp.int64)
    valid = jnp.arange(M, dtype=jnp.int32)[None, :] < counts[:, None]
    return candidates, valid


# speedup vs baseline: 855.3179x; 2.0185x over previous
import functools

import jax
import jax.numpy as jnp
from jax import lax
from jax.experimental import pallas as pl
from jax.experimental.pallas import tpu as pltpu
from jax.experimental.pallas import tpu_sc as plsc

P = 32
E = 50000
F = 1600000
M = 64
N = 65536
PE1 = P * E + 1

VR = F // 128

NC = 2
NS = 16
NW = NC * NS
QPW = N // NW
CH = 64
NCHUNK = QPW // CH
LANES = 16
NG = CH // LANES
PO_OFF = 128 * 128


def _sc_enumerate(preds32, bounds32, dirs32, ps_off32, po_off32, psv, pov):
    mesh = plsc.VectorSubcoreMesh(core_axis_name="c", subcore_axis_name="s")

    @functools.partial(
        pl.kernel,
        out_type=[
            jax.ShapeDtypeStruct((M, N), jnp.int32),
            jax.ShapeDtypeStruct((N,), jnp.int32),
        ],
        mesh=mesh,
        compiler_params=pltpu.CompilerParams(needs_layout_passes=False),
        scratch_types=[
            pltpu.VMEM((QPW,), jnp.int32),
            pltpu.VMEM((QPW,), jnp.int32),
            pltpu.VMEM((QPW,), jnp.int32),
            pltpu.VMEM((2 * CH,), jnp.int32),
            pltpu.VMEM((2 * CH,), jnp.int32),
            pltpu.VMEM((2 * CH,), jnp.int32),
            pltpu.VMEM((2 * CH,), jnp.int32),
            pltpu.VMEM((CH,), jnp.int32),
            pltpu.VMEM((CH,), jnp.int32),
            pltpu.VMEM((CH,), jnp.int32),
            pltpu.VMEM((4 * CH, 2 * M), jnp.int32),
            pltpu.VMEM((M, 2 * CH), jnp.int32),
            pltpu.SemaphoreType.DMA,
        ],
    )
    def body(p_hbm, b_hbm, d_hbm, pso_hbm, poo_hbm, psv_hbm, pov_hbm,
             cand_hbm, cnt_hbm,
             pv, bv, dv, idx_off, so_ps, so_po, rows_idx, counts_v,
             addr_v, alim_v, bufs, out_v, sem):
        wid = lax.axis_index("s") * NC + lax.axis_index("c")
        qw = wid * jnp.int32(QPW)
        pltpu.sync_copy(p_hbm.at[pl.ds(qw, QPW)], pv)
        pltpu.sync_copy(b_hbm.at[pl.ds(qw, QPW)], bv)
        pltpu.sync_copy(d_hbm.at[pl.ds(qw, QPW)], dv)
        iota = lax.iota(jnp.int32, LANES)

        def chunk(c, carry):
            cb = c * jnp.int32(CH)
            for g in range(NG):
                s = pl.ds(cb + g * LANES, LANES)
                k = pv[s] * jnp.int32(E) + bv[s]
                idx_off[pl.ds(g * LANES, LANES)] = k
                idx_off[pl.ds(CH + g * LANES, LANES)] = k + jnp.int32(1)
            pltpu.async_copy(pso_hbm.at[idx_off], so_ps, sem).wait()
            pltpu.async_copy(poo_hbm.at[idx_off], so_po, sem).wait()
            for g in range(NG):
                sl = pl.ds(g * LANES, LANES)
                sh = pl.ds(CH + g * LANES, LANES)
                dmask = dv[pl.ds(cb + g * LANES, LANES)] != jnp.int32(0)
                st = jnp.where(dmask, so_po[sl], so_ps[sl])
                en = jnp.where(dmask, so_po[sh], so_ps[sh])
                cnt = jnp.minimum(jnp.maximum(en - st, jnp.int32(0)),
                                  jnp.int32(M))
                counts_v[sl] = cnt
                stc = jnp.minimum(st, jnp.int32(F - 1))
                r0 = lax.shift_right_logical(stc, jnp.int32(7))
                r0c = jnp.minimum(r0, jnp.int32(VR - 2))
                lq = iota + jnp.int32(g * LANES)
                plsc.store_scatter(rows_idx, [jnp.int32(2) * lq], r0c)
                plsc.store_scatter(rows_idx,
                                   [jnp.int32(2) * lq + jnp.int32(1)],
                                   r0c + jnp.int32(1))
                base = (jnp.where(dmask, jnp.int32(PO_OFF), jnp.int32(0))
                        + lq * jnp.int32(4 * M)
                        - jnp.int32(128) * r0c)
                addr_v[sl] = base + stc
                alim_v[sl] = base + jnp.int32(F - 1)
            pltpu.async_copy(psv_hbm.at[rows_idx], bufs.at[pl.ds(0, 2 * CH)],
                             sem).wait()
            pltpu.async_copy(pov_hbm.at[rows_idx],
                             bufs.at[pl.ds(2 * CH, 2 * CH)], sem).wait()
            colb = lax.bitwise_and(c, jnp.int32(1)) * jnp.int32(CH)
            for g in range(NG):
                lq = iota + jnp.int32(g * LANES)
                lqo = lq + colb
                alim = alim_v[pl.ds(g * LANES, LANES)]

                def jl(j, acc):
                    a, cj = acc
                    r = lax.shift_right_logical(a, jnp.int32(7))
                    col = lax.bitwise_and(a, jnp.int32(127))
                    vals = plsc.load_gather(bufs, [r, col])
                    plsc.store_scatter(out_v, [cj, lqo], vals)
                    return (jnp.minimum(a + jnp.int32(1), alim),
                            cj + jnp.int32(1))

                lax.fori_loop(0, M, jl,
                              (addr_v[pl.ds(g * LANES, LANES)],
                               jnp.zeros((LANES,), jnp.int32)),
                              unroll=8)
            @pl.when(lax.bitwise_and(c, jnp.int32(1)) == jnp.int32(1))
            def _():
                pltpu.sync_copy(
                    out_v,
                    cand_hbm.at[:, pl.ds(pl.multiple_of(qw + cb - jnp.int32(CH), 2 * CH), 2 * CH)])
            pltpu.sync_copy(counts_v, cnt_hbm.at[pl.ds(qw + cb, CH)])
            return carry

        lax.fori_loop(jnp.int32(0), jnp.int32(NCHUNK), chunk, jnp.int32(0))

    return body(preds32, bounds32, dirs32, ps_off32, po_off32, psv, pov)


def kernel(preds, bound_args, direction, ps_sorted_objs, ps_offsets,
           po_sorted_subjs, po_offsets):
    p32 = preds.astype(jnp.int32)
    b32 = bound_args.astype(jnp.int32)
    d32 = direction.astype(jnp.int32)
    pso32 = ps_offsets.astype(jnp.int32)
    poo32 = po_offsets.astype(jnp.int32)
    psv = ps_sorted_objs.astype(jnp.int32).reshape(VR, 128)
    pov = po_sorted_subjs.astype(jnp.int32).reshape(VR, 128)
    candT, counts = _sc_enumerate(p32, b32, d32, pso32, poo32, psv, pov)
    candidates = candT.T.astype(jnp.uint32).astype(jnp.int64)
    valid = jnp.arange(M, dtype=jnp.int32)[None, :] < counts[:, None]
    return candidates, valid


# speedup vs baseline: 1022.4533x; 1.1954x over previous
import functools

import jax
import jax.numpy as jnp
from jax import lax
from jax.experimental import pallas as pl
from jax.experimental.pallas import tpu as pltpu
from jax.experimental.pallas import tpu_sc as plsc

P = 32
E = 50000
F = 1600000
M = 64
N = 65536
PE1 = P * E + 1

VR = F // 128

NC = 2
NS = 16
NW = NC * NS
QPW = N // NW
CH = 64
NCHUNK = QPW // CH
LANES = 16
NG = CH // LANES
PO_OFF = 128 * 128


def _sc_enumerate(preds32, bounds32, dirs32, ps_off32, po_off32, psv, pov):
    mesh = plsc.VectorSubcoreMesh(core_axis_name="c", subcore_axis_name="s")

    @functools.partial(
        pl.kernel,
        out_type=[
            jax.ShapeDtypeStruct((M, N), jnp.int32),
            jax.ShapeDtypeStruct((N,), jnp.int32),
        ],
        mesh=mesh,
        compiler_params=pltpu.CompilerParams(needs_layout_passes=False),
        scratch_types=[
            pltpu.VMEM((QPW,), jnp.int32),
            pltpu.VMEM((QPW,), jnp.int32),
            pltpu.VMEM((QPW,), jnp.int32),
            pltpu.VMEM((QPW,), jnp.int32),
            pltpu.VMEM((2 * CH,), jnp.int32),
            pltpu.VMEM((2 * CH,), jnp.int32),
            pltpu.VMEM((2 * CH,), jnp.int32),
            pltpu.VMEM((2 * CH,), jnp.int32),
            pltpu.VMEM((2 * CH,), jnp.int32),
            pltpu.VMEM((CH,), jnp.int32),
            pltpu.VMEM((CH,), jnp.int32),
            pltpu.VMEM((CH,), jnp.int32),
            pltpu.VMEM((CH,), jnp.int32),
            pltpu.VMEM((4 * CH, 2 * M), jnp.int32),
            pltpu.VMEM((4 * CH, 2 * M), jnp.int32),
            pltpu.VMEM((M, 2 * CH), jnp.int32),
            pltpu.SemaphoreType.DMA,
            pltpu.SemaphoreType.DMA,
            pltpu.SemaphoreType.DMA,
        ],
    )
    def body(p_hbm, b_hbm, d_hbm, pso_hbm, poo_hbm, psv_hbm, pov_hbm,
             cand_hbm, cnt_hbm,
             pv, bv, dv, counts_v, idx_off, so_ps, so_po,
             rows0, rows1, addr0, addr1, alim0, alim1, bufs0, bufs1,
             out_v, sem_off, semr0, semr1):
        wid = lax.axis_index("s") * NC + lax.axis_index("c")
        qw = wid * jnp.int32(QPW)
        pltpu.sync_copy(p_hbm.at[pl.ds(qw, QPW)], pv)
        pltpu.sync_copy(b_hbm.at[pl.ds(qw, QPW)], bv)
        pltpu.sync_copy(d_hbm.at[pl.ds(qw, QPW)], dv)
        iota = lax.iota(jnp.int32, LANES)
        rows = (rows0, rows1)
        addr = (addr0, addr1)
        alim = (alim0, alim1)
        bufs = (bufs0, bufs1)
        semr = (semr0, semr1)

        def issue_off(c):
            cb = c * jnp.int32(CH)
            for g in range(NG):
                s = pl.ds(cb + g * LANES, LANES)
                k = pv[s] * jnp.int32(E) + bv[s]
                idx_off[pl.ds(g * LANES, LANES)] = k
                idx_off[pl.ds(CH + g * LANES, LANES)] = k + jnp.int32(1)
            pltpu.async_copy(pso_hbm.at[idx_off], so_ps, sem_off)
            pltpu.async_copy(poo_hbm.at[idx_off], so_po, sem_off)

        def wait_off():
            pltpu.make_async_copy(pso_hbm.at[idx_off], so_ps, sem_off).wait()
            pltpu.make_async_copy(poo_hbm.at[idx_off], so_po, sem_off).wait()

        def compute(c, par):
            cb = c * jnp.int32(CH)
            for g in range(NG):
                sl = pl.ds(g * LANES, LANES)
                sh = pl.ds(CH + g * LANES, LANES)
                dmask = dv[pl.ds(cb + g * LANES, LANES)] != jnp.int32(0)
                st = jnp.where(dmask, so_po[sl], so_ps[sl])
                en = jnp.where(dmask, so_po[sh], so_ps[sh])
                cnt = jnp.minimum(jnp.maximum(en - st, jnp.int32(0)),
                                  jnp.int32(M))
                counts_v[pl.ds(cb + g * LANES, LANES)] = cnt
                stc = jnp.minimum(st, jnp.int32(F - 1))
                r0 = lax.shift_right_logical(stc, jnp.int32(7))
                r0c = jnp.minimum(r0, jnp.int32(VR - 2))
                lq = iota + jnp.int32(g * LANES)
                plsc.store_scatter(rows[par], [jnp.int32(2) * lq], r0c)
                plsc.store_scatter(rows[par],
                                   [jnp.int32(2) * lq + jnp.int32(1)],
                                   r0c + jnp.int32(1))
                base = (jnp.where(dmask, jnp.int32(PO_OFF), jnp.int32(0))
                        + lq * jnp.int32(4 * M)
                        - jnp.int32(128) * r0c)
                addr[par][sl] = base + stc
                alim[par][sl] = base + jnp.int32(F - 1)

        def issue_rows(par):
            pltpu.async_copy(psv_hbm.at[rows[par]],
                             bufs[par].at[pl.ds(0, 2 * CH)], semr[par])
            pltpu.async_copy(pov_hbm.at[rows[par]],
                             bufs[par].at[pl.ds(2 * CH, 2 * CH)], semr[par])

        def wait_rows(par):
            pltpu.make_async_copy(psv_hbm.at[rows[par]],
                                  bufs[par].at[pl.ds(0, 2 * CH)],
                                  semr[par]).wait()
            pltpu.make_async_copy(pov_hbm.at[rows[par]],
                                  bufs[par].at[pl.ds(2 * CH, 2 * CH)],
                                  semr[par]).wait()

        def extract(par, colhalf):
            b = bufs[par]
            for g in range(NG):
                lqo = iota + jnp.int32(g * LANES + colhalf * CH)
                al = alim[par][pl.ds(g * LANES, LANES)]

                def jl(j, acc):
                    a, cj = acc
                    r = lax.shift_right_logical(a, jnp.int32(7))
                    col = lax.bitwise_and(a, jnp.int32(127))
                    vals = plsc.load_gather(b, [r, col])
                    plsc.store_scatter(out_v, [cj, lqo], vals)
                    return (jnp.minimum(a + jnp.int32(1), al),
                            cj + jnp.int32(1))

                lax.fori_loop(0, M, jl,
                              (addr[par][pl.ds(g * LANES, LANES)],
                               jnp.zeros((LANES,), jnp.int32)),
                              unroll=8)

        def out_dma(c_low):
            start = pl.multiple_of(qw + c_low * jnp.int32(CH), 2 * CH)
            pltpu.sync_copy(out_v, cand_hbm.at[:, pl.ds(start, 2 * CH)])

        issue_off(jnp.int32(0))

        def pipe(i, carry):
            c0 = jnp.int32(2) * i
            c1 = c0 + jnp.int32(1)
            wait_off()
            compute(c0, 0)
            issue_rows(0)
            issue_off(c1)

            @pl.when(i > jnp.int32(0))
            def _():
                wait_rows(1)
                extract(1, 1)
                out_dma(c0 - jnp.int32(2))

            wait_off()
            compute(c1, 1)
            issue_rows(1)

            @pl.when(c1 + jnp.int32(1) < jnp.int32(NCHUNK))
            def _():
                issue_off(c1 + jnp.int32(1))

            wait_rows(0)
            extract(0, 0)
            return carry

        lax.fori_loop(jnp.int32(0), jnp.int32(NCHUNK // 2), pipe,
                      jnp.int32(0))
        wait_rows(1)
        extract(1, 1)
        out_dma(jnp.int32(NCHUNK - 2))
        pltpu.sync_copy(counts_v, cnt_hbm.at[pl.ds(qw, QPW)])

    return body(preds32, bounds32, dirs32, ps_off32, po_off32, psv, pov)


def kernel(preds, bound_args, direction, ps_sorted_objs, ps_offsets,
           po_sorted_subjs, po_offsets):
    p32 = preds.astype(jnp.int32)
    b32 = bound_args.astype(jnp.int32)
    d32 = direction.astype(jnp.int32)
    pso32 = ps_offsets.astype(jnp.int32)
    poo32 = po_offsets.astype(jnp.int32)
    psv = ps_sorted_objs.astype(jnp.int32).reshape(VR, 128)
    pov = po_sorted_subjs.astype(jnp.int32).reshape(VR, 128)
    candT, counts = _sc_enumerate(p32, b32, d32, pso32, poo32, psv, pov)
    candidates = candT.T.astype(jnp.uint32).astype(jnp.int64)
    valid = jnp.arange(M, dtype=jnp.int32)[None, :] < counts[:, None]
    return candidates, valid


# speedup vs baseline: 1033.9865x; 1.0113x over previous
import functools

import jax
import jax.numpy as jnp
from jax import lax
from jax.experimental import pallas as pl
from jax.experimental.pallas import tpu as pltpu
from jax.experimental.pallas import tpu_sc as plsc

P = 32
E = 50000
F = 1600000
M = 64
N = 65536
PE1 = P * E + 1

VR = F // 128

NC = 2
NS = 16
NW = NC * NS
QPW = N // NW
CH = 64
NCHUNK = QPW // CH
LANES = 16
NG = CH // LANES
PO_OFF = 128 * 128

_MESH = plsc.VectorSubcoreMesh(core_axis_name="c", subcore_axis_name="s")
_PARAMS = pltpu.CompilerParams(needs_layout_passes=False)


def _sc_offsets(preds32, bounds32, dirs32, ps_off32, po_off32):
    @functools.partial(
        pl.kernel,
        out_type=[
            jax.ShapeDtypeStruct((N,), jnp.int32),
            jax.ShapeDtypeStruct((N,), jnp.int32),
        ],
        mesh=_MESH,
        compiler_params=_PARAMS,
        scratch_types=[
            pltpu.VMEM((QPW,), jnp.int32),
            pltpu.VMEM((QPW,), jnp.int32),
            pltpu.VMEM((QPW,), jnp.int32),
            pltpu.VMEM((QPW,), jnp.int32),
            pltpu.VMEM((QPW,), jnp.int32),
            pltpu.VMEM((2 * CH,), jnp.int32),
            pltpu.VMEM((2 * CH,), jnp.int32),
            pltpu.VMEM((2 * CH,), jnp.int32),
            pltpu.VMEM((2 * CH,), jnp.int32),
            pltpu.VMEM((2 * CH,), jnp.int32),
            pltpu.VMEM((2 * CH,), jnp.int32),
            pltpu.SemaphoreType.DMA,
            pltpu.SemaphoreType.DMA,
        ],
    )
    def body(p_hbm, b_hbm, d_hbm, pso_hbm, poo_hbm, st_hbm, cnt_hbm,
             pv, bv, dv, st_acc, cnt_acc,
             idx0, idx1, sops0, sops1, sopo0, sopo1, sem0, sem1):
        wid = lax.axis_index("s") * NC + lax.axis_index("c")
        qw = wid * jnp.int32(QPW)
        pltpu.sync_copy(p_hbm.at[pl.ds(qw, QPW)], pv)
        pltpu.sync_copy(b_hbm.at[pl.ds(qw, QPW)], bv)
        pltpu.sync_copy(d_hbm.at[pl.ds(qw, QPW)], dv)
        idx = (idx0, idx1)
        sops = (sops0, sops1)
        sopo = (sopo0, sopo1)
        sem = (sem0, sem1)

        def issue(c, par):
            cb = c * jnp.int32(CH)
            for g in range(NG):
                s = pl.ds(cb + g * LANES, LANES)
                k = pv[s] * jnp.int32(E) + bv[s]
                idx[par][pl.ds(g * LANES, LANES)] = k
                idx[par][pl.ds(CH + g * LANES, LANES)] = k + jnp.int32(1)
            pltpu.async_copy(pso_hbm.at[idx[par]], sops[par], sem[par])
            pltpu.async_copy(poo_hbm.at[idx[par]], sopo[par], sem[par])

        def wait(par):
            pltpu.make_async_copy(pso_hbm.at[idx[par]], sops[par],
                                  sem[par]).wait()
            pltpu.make_async_copy(poo_hbm.at[idx[par]], sopo[par],
                                  sem[par]).wait()

        def consume(c, par):
            cb = c * jnp.int32(CH)
            for g in range(NG):
                sl = pl.ds(g * LANES, LANES)
                sh = pl.ds(CH + g * LANES, LANES)
                dmask = dv[pl.ds(cb + g * LANES, LANES)] != jnp.int32(0)
                st = jnp.where(dmask, sopo[par][sl], sops[par][sl])
                en = jnp.where(dmask, sopo[par][sh], sops[par][sh])
                cnt = jnp.minimum(jnp.maximum(en - st, jnp.int32(0)),
                                  jnp.int32(M))
                st_acc[pl.ds(cb + g * LANES, LANES)] = st
                cnt_acc[pl.ds(cb + g * LANES, LANES)] = cnt

        issue(jnp.int32(0), 0)

        def pipe(i, carry):
            c0 = jnp.int32(2) * i
            c1 = c0 + jnp.int32(1)
            issue(c1, 1)
            wait(0)
            consume(c0, 0)

            @pl.when(c1 + jnp.int32(1) < jnp.int32(NCHUNK))
            def _():
                issue(c1 + jnp.int32(1), 0)

            wait(1)
            consume(c1, 1)
            return carry

        lax.fori_loop(jnp.int32(0), jnp.int32(NCHUNK // 2), pipe,
                      jnp.int32(0))
        pltpu.sync_copy(st_acc, st_hbm.at[pl.ds(qw, QPW)])
        pltpu.sync_copy(cnt_acc, cnt_hbm.at[pl.ds(qw, QPW)])

    return body(preds32, bounds32, dirs32, ps_off32, po_off32)


def _sc_rows(dirs32, starts, psv, pov):
    @functools.partial(
        pl.kernel,
        out_type=jax.ShapeDtypeStruct((M, N), jnp.int32),
        mesh=_MESH,
        compiler_params=_PARAMS,
        scratch_types=[
            pltpu.VMEM((QPW,), jnp.int32),
            pltpu.VMEM((QPW,), jnp.int32),
            pltpu.VMEM((2 * CH,), jnp.int32),
            pltpu.VMEM((2 * CH,), jnp.int32),
            pltpu.VMEM((CH,), jnp.int32),
            pltpu.VMEM((CH,), jnp.int32),
            pltpu.VMEM((CH,), jnp.int32),
            pltpu.VMEM((CH,), jnp.int32),
            pltpu.VMEM((4 * CH, 2 * M), jnp.int32),
            pltpu.VMEM((4 * CH, 2 * M), jnp.int32),
            pltpu.VMEM((M, 2 * CH), jnp.int32),
            pltpu.SemaphoreType.DMA,
            pltpu.SemaphoreType.DMA,
        ],
    )
    def body(d_hbm, st_hbm, psv_hbm, pov_hbm, cand_hbm,
             dv, stv, rows0, rows1, addr0, addr1, alim0, alim1,
             bufs0, bufs1, out_v, semr0, semr1):
        wid = lax.axis_index("s") * NC + lax.axis_index("c")
        qw = wid * jnp.int32(QPW)
        pltpu.sync_copy(d_hbm.at[pl.ds(qw, QPW)], dv)
        pltpu.sync_copy(st_hbm.at[pl.ds(qw, QPW)], stv)
        iota = lax.iota(jnp.int32, LANES)
        rows = (rows0, rows1)
        addr = (addr0, addr1)
        alim = (alim0, alim1)
        bufs = (bufs0, bufs1)
        semr = (semr0, semr1)

        def compute(c, par):
            cb = c * jnp.int32(CH)
            for g in range(NG):
                sl = pl.ds(g * LANES, LANES)
                dmask = dv[pl.ds(cb + g * LANES, LANES)] != jnp.int32(0)
                st = stv[pl.ds(cb + g * LANES, LANES)]
                stc = jnp.minimum(st, jnp.int32(F - 1))
                r0 = lax.shift_right_logical(stc, jnp.int32(7))
                r0c = jnp.minimum(r0, jnp.int32(VR - 2))
                lq = iota + jnp.int32(g * LANES)
                plsc.store_scatter(rows[par], [jnp.int32(2) * lq], r0c)
                plsc.store_scatter(rows[par],
                                   [jnp.int32(2) * lq + jnp.int32(1)],
                                   r0c + jnp.int32(1))
                base = (jnp.where(dmask, jnp.int32(PO_OFF), jnp.int32(0))
                        + lq * jnp.int32(4 * M)
                        - jnp.int32(128) * r0c)
                addr[par][sl] = base + stc
                alim[par][sl] = base + jnp.int32(F - 1)

        def issue_rows(par):
            pltpu.async_copy(psv_hbm.at[rows[par]],
                             bufs[par].at[pl.ds(0, 2 * CH)], semr[par])
            pltpu.async_copy(pov_hbm.at[rows[par]],
                             bufs[par].at[pl.ds(2 * CH, 2 * CH)], semr[par])

        def wait_rows(par):
            pltpu.make_async_copy(psv_hbm.at[rows[par]],
                                  bufs[par].at[pl.ds(0, 2 * CH)],
                                  semr[par]).wait()
            pltpu.make_async_copy(pov_hbm.at[rows[par]],
                                  bufs[par].at[pl.ds(2 * CH, 2 * CH)],
                                  semr[par]).wait()

        def extract(par, colhalf):
            b = bufs[par]
            for g in range(NG):
                lqo = iota + jnp.int32(g * LANES + colhalf * CH)
                al = alim[par][pl.ds(g * LANES, LANES)]

                def jl(j, acc):
                    a, cj = acc
                    r = lax.shift_right_logical(a, jnp.int32(7))
                    col = lax.bitwise_and(a, jnp.int32(127))
                    vals = plsc.load_gather(b, [r, col])
                    plsc.store_scatter(out_v, [cj, lqo], vals)
                    return (jnp.minimum(a + jnp.int32(1), al),
                            cj + jnp.int32(1))

                lax.fori_loop(0, M, jl,
                              (addr[par][pl.ds(g * LANES, LANES)],
                               jnp.zeros((LANES,), jnp.int32)),
                              unroll=8)

        def out_dma(c_low):
            start = pl.multiple_of(qw + c_low * jnp.int32(CH), 2 * CH)
            pltpu.sync_copy(out_v, cand_hbm.at[:, pl.ds(start, 2 * CH)])

        compute(jnp.int32(0), 0)
        issue_rows(0)

        def pipe(i, carry):
            c0 = jnp.int32(2) * i
            c1 = c0 + jnp.int32(1)
            compute(c1, 1)
            issue_rows(1)

            @pl.when(i > jnp.int32(0))
            def _():
                out_dma(c0 - jnp.int32(2))

            wait_rows(0)
            extract(0, 0)

            @pl.when(c1 + jnp.int32(1) < jnp.int32(NCHUNK))
            def _():
                compute(c1 + jnp.int32(1), 0)
                issue_rows(0)

            wait_rows(1)
            extract(1, 1)
            return carry

        lax.fori_loop(jnp.int32(0), jnp.int32(NCHUNK // 2), pipe,
                      jnp.int32(0))
        out_dma(jnp.int32(NCHUNK - 2))

    return body(dirs32, starts, psv, pov)


def kernel(preds, bound_args, direction, ps_sorted_objs, ps_offsets,
           po_sorted_subjs, po_offsets):
    p32 = preds.astype(jnp.int32)
    b32 = bound_args.astype(jnp.int32)
    d32 = direction.astype(jnp.int32)
    pso32 = ps_offsets.astype(jnp.int32)
    poo32 = po_offsets.astype(jnp.int32)
    starts, counts = _sc_offsets(p32, b32, d32, pso32, poo32)
    psv = ps_sorted_objs.astype(jnp.int32).reshape(VR, 128)
    pov = po_sorted_subjs.astype(jnp.int32).reshape(VR, 128)
    candT = _sc_rows(d32, starts, psv, pov)
    candidates = candT.T.astype(jnp.uint32).astype(jnp.int64)
    valid = jnp.arange(M, dtype=jnp.int32)[None, :] < counts[:, None]
    return candidates, valid
